# baseline (speedup 1.0000x reference)
import functools

import jax
import jax.numpy as jnp
from jax import lax
from jax.experimental import pallas as pl
from jax.experimental.pallas import tpu as pltpu
from jax.experimental.pallas import tpu_sc as plsc

_LANES = 16
_CORES = 2
_SUBCORES = 16
_TILES = _CORES * _SUBCORES
_CHUNK = 128
_ROW = 8
_TC_BLK = 2048
_NBUF = 6


def _build_table_tc(xp, pp, W, b, head_w, n_pad):
    grid = n_pad // _TC_BLK

    def body(x_ref, pos_ref, w_ref, b_ref, hw_ref, tab_ref):
        w2 = jnp.dot(w_ref[...], hw_ref[...])
        cc = jnp.dot(b_ref[...], hw_ref[...])
        g = jnp.dot(x_ref[...], w2) + cc
        z = jnp.zeros((_TC_BLK, 3), jnp.float32)
        tab_ref[...] = jnp.concatenate(
            [pos_ref[...], x_ref[...][:, 1:3], g, z], axis=1)

    return pl.pallas_call(
        body,
        grid=(grid,),
        in_specs=[
            pl.BlockSpec((_TC_BLK, 4), lambda i: (i, 0)),
            pl.BlockSpec((_TC_BLK, 2), lambda i: (i, 0)),
            pl.BlockSpec((4, 64), lambda i: (0, 0)),
            pl.BlockSpec((1, 64), lambda i: (0, 0)),
            pl.BlockSpec((64, 1), lambda i: (0, 0)),
        ],
        out_specs=pl.BlockSpec((_TC_BLK, _ROW), lambda i: (i, 0)),
        out_shape=jax.ShapeDtypeStruct((n_pad, _ROW), jnp.float32),
    )(xp, pp, W, b.reshape(1, 64), head_w)


def _sc_flux(table, pos, ei, n_pad):
    e = ei.shape[1]
    assert e % _CHUNK == 0
    n_chunks = e // _CHUNK
    bc, rem = divmod(n_chunks, _TILES)
    assert bc >= _NBUF
    k_max = bc + (1 if rem else 0)
    outer_n = -(-k_max // _NBUF)
    nsl = n_pad // _SUBCORES

    mesh = plsc.VectorSubcoreMesh(
        core_axis_name="c", subcore_axis_name="s",
        num_cores=_CORES, num_subcores=_SUBCORES)

    @functools.partial(
        pl.kernel,
        out_type=jax.ShapeDtypeStruct((_TILES, n_pad), jnp.float32),
        mesh=mesh,
        compiler_params=pltpu.CompilerParams(
            needs_layout_passes=False, use_tc_tiling_on_sc=False),
        scratch_types=[
            [pltpu.VMEM((_CHUNK,), jnp.int32) for _ in range(_NBUF)],
            [pltpu.VMEM((_CHUNK,), jnp.int32) for _ in range(_NBUF)],
            [pltpu.VMEM((_CHUNK, _ROW), jnp.float32) for _ in range(_NBUF)],
            [pltpu.VMEM((_CHUNK, 2), jnp.float32) for _ in range(_NBUF)],
            pltpu.VMEM((n_pad,), jnp.float32),
            [pltpu.SemaphoreType.DMA for _ in range(_NBUF)],
            [pltpu.SemaphoreType.DMA for _ in range(_NBUF)],
        ],
    )
    def k(tab_hbm, pos_hbm, ei_hbm, out_hbm,
          sidx, didx, srows, drows, acc, semi, semr):
        cid = lax.axis_index("c")
        sid = lax.axis_index("s")
        wid = cid * _SUBCORES + sid
        cnt = bc + jnp.where(wid < rem, 1, 0)
        base0 = (wid * bc + jnp.minimum(wid, rem)) * _CHUNK

        zero64 = jnp.zeros((_LANES,), jnp.float32)

        def zbody(j, carry):
            base = j * 4 * _LANES
            acc[pl.ds(base, _LANES)] = zero64
            acc[pl.ds(base + _LANES, _LANES)] = zero64
            acc[pl.ds(base + 2 * _LANES, _LANES)] = zero64
            acc[pl.ds(base + 3 * _LANES, _LANES)] = zero64
            return carry

        lax.fori_loop(0, n_pad // (4 * _LANES), zbody, 0)

        lane = lax.iota(jnp.int32, _LANES)
        cols = [jnp.full((_LANES,), c, jnp.int32) for c in range(5)]

        def fire_idx(j, s):
            base = base0 + j * _CHUNK
            pltpu.async_copy(ei_hbm.at[0, pl.ds(base, _CHUNK)], sidx[s], semi[s])
            pltpu.async_copy(ei_hbm.at[1, pl.ds(base, _CHUNK)], didx[s], semi[s])

        def wait_idx(s):
            pltpu.make_async_copy(
                ei_hbm.at[0, pl.ds(0, _CHUNK)], sidx[s], semi[s]).wait()
            pltpu.make_async_copy(
                ei_hbm.at[1, pl.ds(0, _CHUNK)], didx[s], semi[s]).wait()

        def fire_rows(s):
            pltpu.async_copy(tab_hbm.at[sidx[s]], srows[s], semr[s])
            pltpu.async_copy(pos_hbm.at[didx[s]], drows[s], semr[s])

        def wait_rows(s):
            pltpu.make_async_copy(tab_hbm.at[sidx[s]], srows[s], semr[s]).wait()
            pltpu.make_async_copy(pos_hbm.at[didx[s]], drows[s], semr[s]).wait()

        def compute(s):
            for gg in range(_CHUNK // _LANES):
                rows = lane + gg * _LANES
                px_s = plsc.load_gather(srows[s], [rows, cols[0]])
                py_s = plsc.load_gather(srows[s], [rows, cols[1]])
                u_s = plsc.load_gather(srows[s], [rows, cols[2]])
                v_s = plsc.load_gather(srows[s], [rows, cols[3]])
                g_s = plsc.load_gather(srows[s], [rows, cols[4]])
                px_d = plsc.load_gather(drows[s], [rows, cols[0]])
                py_d = plsc.load_gather(drows[s], [rows, cols[1]])
                dx = px_d - px_s
                dy = py_d - py_s
                r2 = dx * dx + dy * dy
                num = jnp.maximum(u_s * dx + v_s * dy, 0.0)
                yi = (plsc.bitcast(r2, jnp.int32) >> 1) + 0x1FBD1DF5
                y = plsc.bitcast(yi, jnp.float32)
                y = 0.5 * (y + r2 / y)
                y = 0.5 * (y + r2 / y)
                y = 0.5 * (y + r2 / y)
                dist = y + 1e-6
                m = num / (dist * dist) * g_s
                d16 = didx[s][pl.ds(gg * _LANES, _LANES)]
                s16 = sidx[s][pl.ds(gg * _LANES, _LANES)]
                plsc.addupdate_scatter(acc, [d16], m)
                plsc.addupdate_scatter(acc, [s16], -m)

        for j in range(4):
            fire_idx(j, j)
        wait_idx(0)
        fire_rows(0)
        wait_idx(1)
        fire_rows(1)

        def outer(i0, carry):
            for b in range(_NBUF):
                i = i0 * _NBUF + b
                s = b
                s1 = (b + 2) % _NBUF
                s2 = (b + 4) % _NBUF

                @pl.when(i + 2 < cnt)
                def _():
                    wait_idx(s1)
                    fire_rows(s1)

                @pl.when(i + 4 < cnt)
                def _():
                    fire_idx(i + 4, s2)

                @pl.when(i < cnt)
                def _():
                    wait_rows(s)
                    compute(s)
            return carry

        lax.fori_loop(0, outer_n, outer, 0)

        pltpu.sync_copy(acc, out_hbm.at[wid])

    return k(table, pos, ei)


def _epilogue_tc(partials, x0p, head_b, n_pad):
    x2 = x0p.reshape(1, n_pad)
    hb = head_b.reshape(1, 1)

    def body(p_ref, x_ref, hb_ref, pm_ref, dl_ref):
        d = jnp.sum(p_ref[...], axis=0, keepdims=True) + hb_ref[...]
        dl_ref[...] = d
        pm_ref[...] = jax.nn.softplus(x_ref[...] + d)

    return pl.pallas_call(
        body,
        out_shape=(jax.ShapeDtypeStruct((1, n_pad), jnp.float32),
                   jax.ShapeDtypeStruct((1, n_pad), jnp.float32)),
    )(partials, x2, hb)


def kernel(x, edge_index, pos, W, b, head_w, head_b):
    n = x.shape[0]
    n_pad = -(-n // _TC_BLK) * _TC_BLK

    xp = jnp.pad(x, ((0, n_pad - n), (0, 0)))
    pp = jnp.pad(pos, ((0, n_pad - n), (0, 0)))

    table = _build_table_tc(xp, pp, W, b, head_w, n_pad)
    partials = _sc_flux(table, pp, edge_index, n_pad)
    pm_p, dl_p = _epilogue_tc(partials, xp[:, 0], head_b, n_pad)

    pm10_next = pm_p.reshape(n_pad, 1)[:n]
    delta_pm10 = dl_p.reshape(n_pad, 1)[:n]
    return (pm10_next, delta_pm10)


# speedup vs baseline: 1.3043x; 1.3043x over previous
import functools

import numpy as np

import jax
import jax.numpy as jnp
from jax import lax
from jax.experimental import pallas as pl
from jax.experimental.pallas import tpu as pltpu
from jax.experimental.pallas import tpu_sc as plsc

_LANES = 16
_CORES = 2
_SUBCORES = 16
_TILES = _CORES * _SUBCORES
_CHUNK = 128
_ROW = 8
_NPB = 32
_NBUF = 6
_HI = jax.lax.Precision.HIGHEST


def _perm_consts():
    a = np.zeros((64, 256), np.float32)
    for i in range(64):
        a[i, (i // 2) * 8 + i % 2] = 1.0
    bcst = np.zeros((128, 256), np.float32)
    for i in range(128):
        s, k = divmod(i, 4)
        if k == 1:
            bcst[i, s * 8 + 2] = 1.0
        elif k == 2:
            bcst[i, s * 8 + 3] = 1.0
    gsel = np.zeros((128, 256), np.float32)
    for i in range(128):
        gsel[i, (i // 4) * 8 + 4] = 1.0
    gcols = np.zeros((1, 256), np.float32)
    gcols[0, 4::8] = 1.0
    return a, bcst, gsel, gcols


_A_PERM, _B_PERM, _G_SEL, _G_COLS = _perm_consts()


def _build_table_tc(xr, pr, W, b, head_w):
    rows = xr.shape[0]

    def body(x_ref, pos_ref, w_ref, b_ref, hw_ref,
             a_ref, bp_ref, gs_ref, gc_ref, tab_ref):
        w2 = jnp.dot(w_ref[...], hw_ref[...], precision=_HI)
        cc = jnp.dot(b_ref[...], hw_ref[...], precision=_HI)
        bmat = bp_ref[...] + gs_ref[...] * jnp.tile(w2, (32, 1))
        tab = (jnp.dot(pos_ref[...], a_ref[...], precision=_HI)
               + jnp.dot(x_ref[...], bmat, precision=_HI)
               + cc * gc_ref[...])
        tab_ref[...] = tab

    return pl.pallas_call(
        body,
        out_shape=jax.ShapeDtypeStruct((rows, _ROW * _NPB), jnp.float32),
    )(xr, pr, W, b.reshape(1, 64), head_w,
      jnp.asarray(_A_PERM), jnp.asarray(_B_PERM),
      jnp.asarray(_G_SEL), jnp.asarray(_G_COLS))


def _sc_flux(tab8, ei2, n_pad):
    n_chunks = ei2.shape[0] // 2
    bc, rem = divmod(n_chunks, _TILES)
    assert bc >= _NBUF
    k_max = bc + (1 if rem else 0)
    outer_n = -(-k_max // _NBUF)

    mesh = plsc.VectorSubcoreMesh(
        core_axis_name="c", subcore_axis_name="s",
        num_cores=_CORES, num_subcores=_SUBCORES)

    @functools.partial(
        pl.kernel,
        out_type=jax.ShapeDtypeStruct((_TILES, n_pad), jnp.float32),
        mesh=mesh,
        compiler_params=pltpu.CompilerParams(
            needs_layout_passes=False, use_tc_tiling_on_sc=False),
        scratch_types=[
            [pltpu.VMEM((_CHUNK,), jnp.int32) for _ in range(_NBUF)],
            [pltpu.VMEM((_CHUNK,), jnp.int32) for _ in range(_NBUF)],
            [pltpu.VMEM((_CHUNK, _ROW), jnp.float32) for _ in range(_NBUF)],
            [pltpu.VMEM((_CHUNK, _ROW), jnp.float32) for _ in range(_NBUF)],
            pltpu.VMEM((n_pad,), jnp.float32),
            [pltpu.SemaphoreType.DMA for _ in range(_NBUF)],
            [pltpu.SemaphoreType.DMA for _ in range(_NBUF)],
        ],
    )
    def k(tab_hbm, ei_hbm, out_hbm,
          sidx, didx, srows, drows, acc, semi, semr):
        cid = lax.axis_index("c")
        sid = lax.axis_index("s")
        wid = cid * _SUBCORES + sid
        cnt = bc + jnp.where(wid < rem, 1, 0)
        chunk0 = wid * bc + jnp.minimum(wid, rem)

        zero16 = jnp.zeros((_LANES,), jnp.float32)

        def zbody(j, carry):
            base = j * 4 * _LANES
            acc[pl.ds(base, _LANES)] = zero16
            acc[pl.ds(base + _LANES, _LANES)] = zero16
            acc[pl.ds(base + 2 * _LANES, _LANES)] = zero16
            acc[pl.ds(base + 3 * _LANES, _LANES)] = zero16
            return carry

        lax.fori_loop(0, n_pad // (4 * _LANES), zbody, 0)

        lane = lax.iota(jnp.int32, _LANES)
        cols = [jnp.full((_LANES,), c, jnp.int32) for c in range(5)]

        def fire_idx(j, s):
            jj = chunk0 + j
            pltpu.async_copy(ei_hbm.at[jj], sidx[s], semi[s])
            pltpu.async_copy(ei_hbm.at[n_chunks + jj], didx[s], semi[s])

        def wait_idx(s):
            pltpu.make_async_copy(ei_hbm.at[0], sidx[s], semi[s]).wait()
            pltpu.make_async_copy(ei_hbm.at[0], didx[s], semi[s]).wait()

        def fire_rows(s):
            pltpu.async_copy(tab_hbm.at[sidx[s]], srows[s], semr[s])
            pltpu.async_copy(tab_hbm.at[didx[s]], drows[s], semr[s])

        def wait_rows(s):
            pltpu.make_async_copy(tab_hbm.at[sidx[s]], srows[s], semr[s]).wait()
            pltpu.make_async_copy(tab_hbm.at[didx[s]], drows[s], semr[s]).wait()

        def compute(s):
            for gg in range(_CHUNK // _LANES):
                rows = lane + gg * _LANES
                px_s = plsc.load_gather(srows[s], [rows, cols[0]])
                py_s = plsc.load_gather(srows[s], [rows, cols[1]])
                u_s = plsc.load_gather(srows[s], [rows, cols[2]])
                v_s = plsc.load_gather(srows[s], [rows, cols[3]])
                g_s = plsc.load_gather(srows[s], [rows, cols[4]])
                px_d = plsc.load_gather(drows[s], [rows, cols[0]])
                py_d = plsc.load_gather(drows[s], [rows, cols[1]])
                dx = px_d - px_s
                dy = py_d - py_s
                r2 = dx * dx + dy * dy
                num = jnp.maximum(u_s * dx + v_s * dy, 0.0)
                yi = (plsc.bitcast(r2, jnp.int32) >> 1) + 0x1FBD1DF5
                y = plsc.bitcast(yi, jnp.float32)
                y = 0.5 * (y + r2 / y)
                y = 0.5 * (y + r2 / y)
                y = 0.5 * (y + r2 / y)
                dist = y + 1e-6
                m = num / (dist * dist) * g_s
                d16 = didx[s][pl.ds(gg * _LANES, _LANES)]
                s16 = sidx[s][pl.ds(gg * _LANES, _LANES)]
                plsc.addupdate_scatter(acc, [d16], m)
                plsc.addupdate_scatter(acc, [s16], -m)

        for j in range(4):
            fire_idx(j, j)
        wait_idx(0)
        fire_rows(0)
        wait_idx(1)
        fire_rows(1)

        def outer(i0, carry):
            for b in range(_NBUF):
                i = i0 * _NBUF + b
                s = b
                s1 = (b + 2) % _NBUF
                s2 = (b + 4) % _NBUF

                @pl.when(i + 2 < cnt)
                def _():
                    wait_idx(s1)
                    fire_rows(s1)

                @pl.when(i + 4 < cnt)
                def _():
                    fire_idx(i + 4, s2)

                @pl.when(i < cnt)
                def _():
                    wait_rows(s)
                    compute(s)
            return carry

        lax.fori_loop(0, outer_n, outer, 0)

        pltpu.sync_copy(acc, out_hbm.at[wid])

    return k(tab8, ei2)


def _epilogue_tc(partials, x0p, head_b, n_pad):
    x2 = x0p.reshape(1, n_pad)
    hb = head_b.reshape(1, 1)

    def body(p_ref, x_ref, hb_ref, pm_ref, dl_ref):
        d = jnp.sum(p_ref[...], axis=0, keepdims=True) + hb_ref[...]
        dl_ref[...] = d
        pm_ref[...] = jax.nn.softplus(x_ref[...] + d)

    return pl.pallas_call(
        body,
        out_shape=(jax.ShapeDtypeStruct((1, n_pad), jnp.float32),
                   jax.ShapeDtypeStruct((1, n_pad), jnp.float32)),
    )(partials, x2, hb)


def kernel(x, edge_index, pos, W, b, head_w, head_b):
    n = x.shape[0]
    e = edge_index.shape[1]
    assert n % _NPB == 0 and e % _CHUNK == 0
    n_pad = -(-n // 128) * 128

    xr = x.reshape(n // _NPB, 4 * _NPB)
    pr = pos.reshape(n // _NPB, 2 * _NPB)
    ei2 = edge_index.reshape(2 * e // _CHUNK, _CHUNK)

    table = _build_table_tc(xr, pr, W, b, head_w)
    tab8 = table.reshape(n, _ROW)
    partials = _sc_flux(tab8, ei2, n_pad)

    x0p = jnp.pad(x[:, 0], (0, n_pad - n))
    pm_p, dl_p = _epilogue_tc(partials, x0p, head_b, n_pad)

    pm10_next = pm_p.reshape(n_pad, 1)[:n]
    delta_pm10 = dl_p.reshape(n_pad, 1)[:n]
    return (pm10_next, delta_pm10)
